# baseline (speedup 1.0000x reference)
import functools

import jax
import jax.numpy as jnp
from jax import lax
from jax.experimental import pallas as pl
from jax.experimental.pallas import tpu as pltpu
from jax.experimental.pallas import tpu_sc as plsc

N = 100000
D = 128
B = 500000

NC = 2
NS = 16
NW = NC * NS
L = 16

C = 128
K = 124
BP = NW * C * K
EPW = C * K
GPC = C // L


def _sc_logits_body(emb_hbm, eidx_hbm, out_hbm,
                    eidx, rows0, rows1, lg, sem0, sem1):
    wid = lax.axis_index("s") * NC + lax.axis_index("c")

    pltpu.sync_copy(eidx_hbm.at[pl.ds(wid * K, K)], eidx)

    bufs = (rows0, rows1)
    sems = (sem0, sem1)

    def gather(k, b, issue):
        start = pltpu.async_copy if issue else pltpu.make_async_copy
        return (
            start(emb_hbm.at[eidx.at[k, 0]],
                  bufs[b].at[pl.ds(0, C)], sems[b]),
            start(emb_hbm.at[eidx.at[k, 1]],
                  bufs[b].at[pl.ds(C, C)], sems[b]),
        )

    for b in range(2):
        gather(b, b, issue=True)

    iota2 = 2 * lax.iota(jnp.int32, L)

    def pair_body(kk, carry):
        for b in range(2):
            k = 2 * kk + b
            for c in gather(k, b, issue=False):
                c.wait()

            @pl.when(kk < (K // 2) - 1)
            def _issue_next():
                gather(k + 2, b, issue=True)

            buf = bufs[b]

            def group_body(g, carry2):
                row_s = 2 * L * g + iota2
                row_d = row_s + 1
                acc = jnp.zeros((L,), jnp.float32)
                for f in range(D):
                    colf = jnp.full((L,), f, jnp.int32)
                    a = plsc.load_gather(buf, [row_s, colf])
                    bb = plsc.load_gather(buf, [row_d, colf])
                    acc = acc + a * bb
                lg[pl.ds(k * C + g * L, L)] = acc
                return carry2

            lax.fori_loop(0, GPC, group_body, 0)
        return carry

    lax.fori_loop(0, K // 2, pair_body, 0)
    pltpu.sync_copy(lg, out_hbm.at[pl.ds(wid * EPW, EPW)])


_sc_logits = functools.partial(
    pl.kernel,
    out_type=jax.ShapeDtypeStruct((BP,), jnp.float32),
    mesh=plsc.VectorSubcoreMesh(
        core_axis_name="c", subcore_axis_name="s",
        num_cores=NC, num_subcores=NS),
    compiler_params=pltpu.CompilerParams(needs_layout_passes=False),
    scratch_types=[
        pltpu.VMEM((K, 2, C), jnp.int32),
        pltpu.VMEM((2 * C, D), jnp.float32),
        pltpu.VMEM((2 * C, D), jnp.float32),
        pltpu.VMEM((EPW,), jnp.float32),
        pltpu.SemaphoreType.DMA,
        pltpu.SemaphoreType.DMA,
    ],
)(_sc_logits_body)


def _bce_body(x_ref, y_ref, o_ref):
    x = x_ref[...]
    y = y_ref[...]
    rows, cols = x.shape
    lin = (lax.broadcasted_iota(jnp.int32, (rows, cols), 0) * cols
           + lax.broadcasted_iota(jnp.int32, (rows, cols), 1))
    elt = jnp.maximum(x, 0.0) - x * y + jnp.log1p(jnp.exp(-jnp.abs(x)))
    elt = jnp.where(lin < B, elt, 0.0)
    o_ref[...] = (jnp.sum(elt) / B).reshape(1, 1)


def kernel(node_emb, edge_index, edge_label):
    pad = BP - B
    eidx = jnp.pad(edge_index, ((0, pad), (0, 0))).reshape(NW * K, 2, C)

    logits = _sc_logits(node_emb, eidx)

    rows = BP // 128
    logits2d = logits.reshape(rows, 128)
    labels2d = jnp.pad(edge_label, (0, pad)).reshape(rows, 128)

    loss = pl.pallas_call(
        _bce_body,
        out_shape=jax.ShapeDtypeStruct((1, 1), jnp.float32),
    )(logits2d, labels2d)
    return loss[0, 0]


# speedup vs baseline: 1.8429x; 1.8429x over previous
import functools

import jax
import jax.numpy as jnp
from jax import lax
from jax.experimental import pallas as pl
from jax.experimental.pallas import tpu as pltpu
from jax.experimental.pallas import tpu_sc as plsc

N = 100000
D = 128
B = 500000

NC = 2
NS = 16
NW = NC * NS
L = 16

C = 128
K = 124
BP = NW * C * K
EPW = C * K
GPC = C // L


def _sc_logits_body(emb_hbm, eidx_hbm, out_hbm,
                    eidx, rows0, rows1, lg, sem0, sem1):
    wid = lax.axis_index("s") * NC + lax.axis_index("c")

    pltpu.sync_copy(eidx_hbm.at[pl.ds(wid * K, K)], eidx)

    bufs = (rows0, rows1)
    sems = (sem0, sem1)

    def gather(k, b, issue):
        start = pltpu.async_copy if issue else pltpu.make_async_copy
        return (
            start(emb_hbm.at[eidx.at[k, 0]],
                  bufs[b].at[pl.ds(0, C)], sems[b]),
            start(emb_hbm.at[eidx.at[k, 1]],
                  bufs[b].at[pl.ds(C, C)], sems[b]),
        )

    for b in range(2):
        gather(b, b, issue=True)

    lane15 = lax.iota(jnp.int32, L) == (L - 1)

    def pair_body(kk, carry):
        for b in range(2):
            k = 2 * kk + b
            for c in gather(k, b, issue=False):
                c.wait()

            @pl.when(kk < (K // 2) - 1)
            def _issue_next():
                gather(k + 2, b, issue=True)

            buf = bufs[b]

            @plsc.parallel_loop(0, C, unroll=4)
            def edge_body(e):
                acc0 = buf[2 * e, pl.ds(0, L)] * buf[2 * e + 1, pl.ds(0, L)]
                acc1 = (buf[2 * e, pl.ds(L, L)]
                        * buf[2 * e + 1, pl.ds(L, L)])
                for f in range(2, D // L, 2):
                    acc0 = acc0 + (buf[2 * e, pl.ds(f * L, L)]
                                   * buf[2 * e + 1, pl.ds(f * L, L)])
                    acc1 = acc1 + (buf[2 * e, pl.ds((f + 1) * L, L)]
                                   * buf[2 * e + 1, pl.ds((f + 1) * L, L)])
                tot = plsc.cumsum(acc0 + acc1)
                plsc.store_scatter(lg, [jnp.full((L,), k * C + e, jnp.int32)],
                                   tot, mask=lane15)
        return carry

    lax.fori_loop(0, K // 2, pair_body, 0)
    pltpu.sync_copy(lg, out_hbm.at[pl.ds(wid * EPW, EPW)])


_sc_logits = functools.partial(
    pl.kernel,
    out_type=jax.ShapeDtypeStruct((BP,), jnp.float32),
    mesh=plsc.VectorSubcoreMesh(
        core_axis_name="c", subcore_axis_name="s",
        num_cores=NC, num_subcores=NS),
    compiler_params=pltpu.CompilerParams(needs_layout_passes=False),
    scratch_types=[
        pltpu.VMEM((K, 2, C), jnp.int32),
        pltpu.VMEM((2 * C, D), jnp.float32),
        pltpu.VMEM((2 * C, D), jnp.float32),
        pltpu.VMEM((EPW,), jnp.float32),
        pltpu.SemaphoreType.DMA,
        pltpu.SemaphoreType.DMA,
    ],
)(_sc_logits_body)


def _bce_body(x_ref, y_ref, o_ref):
    x = x_ref[...]
    y = y_ref[...]
    rows, cols = x.shape
    lin = (lax.broadcasted_iota(jnp.int32, (rows, cols), 0) * cols
           + lax.broadcasted_iota(jnp.int32, (rows, cols), 1))
    elt = jnp.maximum(x, 0.0) - x * y + jnp.log1p(jnp.exp(-jnp.abs(x)))
    elt = jnp.where(lin < B, elt, 0.0)
    o_ref[...] = (jnp.sum(elt) / B).reshape(1, 1)


def kernel(node_emb, edge_index, edge_label):
    pad = BP - B
    eidx = jnp.pad(edge_index, ((0, pad), (0, 0))).reshape(NW * K, 2, C)

    logits = _sc_logits(node_emb, eidx)

    rows = BP // 128
    logits2d = logits.reshape(rows, 128)
    labels2d = jnp.pad(edge_label, (0, pad)).reshape(rows, 128)

    loss = pl.pallas_call(
        _bce_body,
        out_shape=jax.ShapeDtypeStruct((1, 1), jnp.float32),
    )(logits2d, labels2d)
    return loss[0, 0]
